# baseline (speedup 1.0000x reference)
import functools

import jax
import jax.numpy as jnp
from jax import lax
from jax.experimental import pallas as pl
from jax.experimental.pallas import tpu as pltpu
from jax.experimental.pallas import tpu_sc as plsc

BATCH = 16384
NF = 100
NBND = 99
BPAD = 112
NROWS = NF * 101
EMB = 128

NC, NS, L = 2, 16, 16
NW = NC * NS
SPAN = BATCH // NW
BSUB = 16
MID = SPAN // BSUB
ESUB = BSUB * NF
NVEC = ESUB // L
G = 32
NG = ESUB // G
NBUF = 5
NRND = NG // NBUF
SL = NVEC // NRND


def _sc_body(xf, bnd, tbl, out, x_v, b_v, idx0, idx1, rows, tbl_sh, gsems, wsems):
    wid = lax.axis_index("s") * NC + lax.axis_index("c")
    e0w = wid * (SPAN * NF)

    @pl.when(lax.axis_index("s") == 0)
    def _():
        pltpu.sync_copy(tbl, tbl_sh)

    pltpu.sync_copy(bnd, b_v)
    plsc.subcore_barrier()

    iota = lax.iota(jnp.int32, L)

    def compute(v, x_v, idx_v):
        base = v * L
        xv = x_v[pl.ds(base, L)]
        f = lax.rem(base + iota, NF)
        fb = f * BPAD
        lo = jnp.zeros((L,), jnp.int32)
        for p in (64, 32, 16, 8, 4, 2, 1):
            cand = lo + p
            probe = plsc.load_gather(b_v, [fb + cand - 1])
            lo = jnp.where(probe < xv, cand, lo)
        idx_v[v // (G // L), pl.ds(lax.rem(v, G // L) * L, L)] = f * 101 + lo

    pltpu.sync_copy(xf.at[pl.ds(e0w, ESUB)], x_v)

    def compute0(v):
        compute(v, x_v, idx0)

    plsc.parallel_loop(0, NVEC, unroll=5)(compute0)

    def one_mid(m, idxa, idxb):
        e0 = e0w + m * ESUB

        for b in range(NBUF):
            pltpu.async_copy(tbl_sh.at[idxa.at[b]], rows.at[b], gsems.at[b])

        @pl.when(m < MID - 1)
        def _():
            pltpu.sync_copy(xf.at[pl.ds(e0 + ESUB, ESUB)], x_v)

        def move(jj, carry):
            j0 = jj * NBUF
            for b in range(NBUF):
                pltpu.make_async_copy(
                    tbl_sh.at[idxa.at[j0 + b]], rows.at[b], gsems.at[b]
                ).wait()
                pltpu.async_copy(
                    rows.at[b], out.at[pl.ds(e0 + (j0 + b) * G, G)], wsems.at[b]
                )
            for b in range(NBUF):
                pltpu.make_async_copy(
                    rows.at[b], out.at[pl.ds(e0 + (j0 + b) * G, G)], wsems.at[b]
                ).wait()

                @pl.when(jj < NRND - 1)
                def _(b=b):
                    pltpu.async_copy(
                        tbl_sh.at[idxa.at[j0 + NBUF + b]], rows.at[b], gsems.at[b]
                    )

            @pl.when(m < MID - 1)
            def _():
                def compute_n(v):
                    compute(v, x_v, idxb)

                plsc.parallel_loop(jj * SL, (jj + 1) * SL, unroll=5)(compute_n)

            return carry

        lax.fori_loop(0, NRND, move, 0)

    def mid_pair(mm, carry_m):
        one_mid(2 * mm, idx0, idx1)
        one_mid(2 * mm + 1, idx1, idx0)
        return carry_m

    lax.fori_loop(0, MID // 2, mid_pair, 0)


def kernel(x, boundaries, tables):
    xf = x.reshape(BATCH * NF)
    bnd = jnp.concatenate(
        [boundaries, jnp.full((NF, BPAD - NBND), jnp.inf, jnp.float32)], axis=1
    ).reshape(NF * BPAD)
    tbl = tables.reshape(NROWS, EMB)

    mesh = plsc.VectorSubcoreMesh(core_axis_name="c", subcore_axis_name="s")
    run = functools.partial(
        pl.kernel,
        mesh=mesh,
        out_type=jax.ShapeDtypeStruct((BATCH * NF, EMB), jnp.float32),
        scratch_types=[
            pltpu.VMEM((ESUB,), jnp.float32),
            pltpu.VMEM((NF * BPAD,), jnp.float32),
            pltpu.VMEM((NG, G), jnp.int32),
            pltpu.VMEM((NG, G), jnp.int32),
            pltpu.VMEM((NBUF, G, EMB), jnp.float32),
            pltpu.VMEM_SHARED((NROWS, EMB), jnp.float32),
            pltpu.SemaphoreType.DMA((NBUF,)),
            pltpu.SemaphoreType.DMA((NBUF,)),
        ],
        compiler_params=pltpu.CompilerParams(needs_layout_passes=False),
    )(_sc_body)
    out = run(xf, bnd, tbl)
    return out.reshape(BATCH, NF * EMB)


# speedup vs baseline: 1.0616x; 1.0616x over previous
import functools

import jax
import jax.numpy as jnp
from jax import lax
from jax.experimental import pallas as pl
from jax.experimental.pallas import tpu as pltpu
from jax.experimental.pallas import tpu_sc as plsc

BATCH = 16384
NF = 100
NBND = 99
BPAD = 112
NROWS = NF * 101
EMB = 128

NC, NS, L = 2, 16, 16
NW = NC * NS
SPAN = BATCH // NW
BSUB = 16
MID = SPAN // BSUB
ESUB = BSUB * NF
NVEC = ESUB // L
G = 32
NG = ESUB // G
NBUF = 5
NRND = NG // NBUF
SL = NVEC // NRND


def _sc_body(xf, bnd, tbl, out, x_v, b_v, idx0, idx1, rows, tbl_sh, gsems, wsems):
    wid = lax.axis_index("s") * NC + lax.axis_index("c")
    e0w = wid * (SPAN * NF)

    @pl.when(lax.axis_index("s") == 0)
    def _():
        pltpu.sync_copy(tbl, tbl_sh)

    pltpu.sync_copy(bnd, b_v)
    plsc.subcore_barrier()

    iota = lax.iota(jnp.int32, L)

    def compute(v, x_v, idx_v):
        base = v * L
        xv = x_v[pl.ds(base, L)]
        f = lax.rem(base + iota, NF)
        fb = f * BPAD
        lo = jnp.zeros((L,), jnp.int32)
        for p in (64, 32, 16, 8, 4, 2, 1):
            cand = lo + p
            probe = plsc.load_gather(b_v, [fb + cand - 1])
            lo = jnp.where(probe < xv, cand, lo)
        idx_v[v // (G // L), pl.ds(lax.rem(v, G // L) * L, L)] = f * 101 + lo

    pltpu.sync_copy(xf.at[pl.ds(e0w, ESUB)], x_v)

    def compute0(v):
        compute(v, x_v, idx0)

    plsc.parallel_loop(0, NVEC, unroll=4)(compute0)

    def one_mid(m, idxa, idxb):
        e0 = e0w + m * ESUB

        for b in range(NBUF):
            pltpu.async_copy(tbl_sh.at[idxa.at[b]], rows.at[b], gsems.at[b])

        @pl.when(m < MID - 1)
        def _():
            pltpu.sync_copy(xf.at[pl.ds(e0 + ESUB, ESUB)], x_v)

        def move(jj, carry):
            j0 = jj * NBUF
            for b in range(NBUF):
                pltpu.make_async_copy(
                    tbl_sh.at[idxa.at[j0 + b]], rows.at[b], gsems.at[b]
                ).wait()
                pltpu.async_copy(
                    rows.at[b], out.at[pl.ds(e0 + (j0 + b) * G, G)], wsems.at[b]
                )
            for b in range(NBUF):
                pltpu.make_async_copy(
                    rows.at[b], out.at[pl.ds(e0 + (j0 + b) * G, G)], wsems.at[b]
                ).wait()

                @pl.when(jj < NRND - 1)
                def _(b=b):
                    pltpu.async_copy(
                        tbl_sh.at[idxa.at[j0 + NBUF + b]], rows.at[b], gsems.at[b]
                    )

            @pl.when(m < MID - 1)
            def _():
                def compute_n(v):
                    compute(v, x_v, idxb)

                plsc.parallel_loop(jj * SL, (jj + 1) * SL, unroll=2)(compute_n)

            return carry

        lax.fori_loop(0, NRND, move, 0)

    def mid_pair(mm, carry_m):
        one_mid(2 * mm, idx0, idx1)
        one_mid(2 * mm + 1, idx1, idx0)
        return carry_m

    lax.fori_loop(0, MID // 2, mid_pair, 0)


def kernel(x, boundaries, tables):
    xf = x.reshape(BATCH * NF)
    bnd = jnp.concatenate(
        [boundaries, jnp.full((NF, BPAD - NBND), jnp.inf, jnp.float32)], axis=1
    ).reshape(NF * BPAD)
    tbl = tables.reshape(NROWS, EMB)

    mesh = plsc.VectorSubcoreMesh(core_axis_name="c", subcore_axis_name="s")
    run = functools.partial(
        pl.kernel,
        mesh=mesh,
        out_type=jax.ShapeDtypeStruct((BATCH * NF, EMB), jnp.float32),
        scratch_types=[
            pltpu.VMEM((ESUB,), jnp.float32),
            pltpu.VMEM((NF * BPAD,), jnp.float32),
            pltpu.VMEM((NG, G), jnp.int32),
            pltpu.VMEM((NG, G), jnp.int32),
            pltpu.VMEM((NBUF, G, EMB), jnp.float32),
            pltpu.VMEM_SHARED((NROWS, EMB), jnp.float32),
            pltpu.SemaphoreType.DMA((NBUF,)),
            pltpu.SemaphoreType.DMA((NBUF,)),
        ],
        compiler_params=pltpu.CompilerParams(needs_layout_passes=False),
    )(_sc_body)
    out = run(xf, bnd, tbl)
    return out.reshape(BATCH, NF * EMB)
